# baseline (speedup 1.0000x reference)
import functools

import jax
import jax.numpy as jnp
from jax import lax
from jax.experimental import pallas as pl
from jax.experimental.pallas import tpu as pltpu
from jax.experimental.pallas import tpu_sc as plsc

DIM = 16
ALPHA = 0.001
LANES = 16
WIN = 128
HALF = 8
SLOTS = 3


def _dot_kernel(n_batch, n_workers, ij_hbm, pt_hbm, mt_hbm, out_hbm,
                ij_v, iv_v, jv_v, out_v, pw, mw, *sems):
    bpw = n_batch // n_workers
    n_blocks = bpw // LANES
    n_halves = 2 * n_blocks
    wid = lax.axis_index("s") * 2 + lax.axis_index("c")
    base = wid * bpw
    sem_p = sems[:SLOTS]
    sem_m = sems[SLOTS:]

    pltpu.sync_copy(ij_hbm.at[pl.ds(2 * base, 2 * bpw)], ij_v)

    def unzip_block(blk, _):
        b0 = blk * LANES
        flat = 2 * (b0 + lax.iota(jnp.int32, LANES))
        iv_v[pl.ds(b0, LANES)] = plsc.load_gather(ij_v, [flat])
        jv_v[pl.ds(b0, LANES)] = plsc.load_gather(ij_v, [flat + 1])
        return 0

    lax.fori_loop(0, n_blocks, unzip_block, 0, unroll=4)

    def win_off(idx_scalar):
        return pl.multiple_of((idx_scalar >> 7) << 7, WIN)

    rows = lax.iota(jnp.int32, LANES)

    def half_indices(h):
        iv8 = iv_v[pl.ds(h * HALF, LANES)]
        jv8 = jv_v[pl.ds(h * HALF, LANES)]
        return iv8, jv8

    def fire_half(h, slot):
        iv8, jv8 = half_indices(h)
        for u in range(HALF):
            ci = win_off(iv8[u])
            cj = win_off(jv8[u])
            pltpu.async_copy(
                pt_hbm.at[:, pl.ds(ci, WIN)], pw.at[slot * HALF + u],
                sem_p[slot])
            pltpu.async_copy(
                mt_hbm.at[:, pl.ds(cj, WIN)], mw.at[slot * HALF + u],
                sem_m[slot])

    def drain_half(slot):
        for u in range(HALF):
            pltpu.make_async_copy(
                pt_hbm.at[:, pl.ds(0, WIN)], pw.at[slot * HALF + u],
                sem_p[slot]).wait()
            pltpu.make_async_copy(
                mt_hbm.at[:, pl.ds(0, WIN)], mw.at[slot * HALF + u],
                sem_m[slot]).wait()

    def compute_store_half(h, slot):
        iv8, jv8 = half_indices(h)
        res = jnp.zeros((LANES,), jnp.float32)
        for u in range(HALF):
            oi = iv8[u] - win_off(iv8[u])
            oj = jv8[u] - win_off(jv8[u])
            pvec = plsc.load_gather(
                pw.at[slot * HALF + u], [rows, jnp.broadcast_to(oi, (LANES,))])
            mvec = plsc.load_gather(
                mw.at[slot * HALF + u], [rows, jnp.broadcast_to(oj, (LANES,))])
            s = jnp.sum(pvec * mvec)
            res = jnp.where(rows == u, s, res)
        out_v[pl.ds(h * HALF, LANES)] = res * jnp.float32(ALPHA)

    fire_half(0, 0)
    fire_half(1, 1)

    def body(g, _):
        for k in range(SLOTS):
            h = g * SLOTS + k
            drain_half(k)
            compute_store_half(h, k)

            @pl.when(h + 2 < n_halves)
            def _():
                fire_half(h + 2, (k + 2) % SLOTS)
        return 0

    lax.fori_loop(0, n_halves // SLOTS, body, 0)
    drain_half((n_halves - 1) % SLOTS)
    compute_store_half(n_halves - 1, (n_halves - 1) % SLOTS)

    pltpu.sync_copy(out_v.at[pl.ds(0, bpw)], out_hbm.at[pl.ds(base, bpw)])


def kernel(ij, P, M):
    ij_flat = ij.astype(jnp.int32).reshape(-1)
    pt = P.T
    mt = M.T
    n_batch = ij.shape[0]
    info = plsc.get_sparse_core_info()
    n_workers = info.num_cores * info.num_subcores
    bpw = n_batch // n_workers

    mesh = plsc.VectorSubcoreMesh(core_axis_name="c", subcore_axis_name="s")
    run = pl.kernel(
        functools.partial(_dot_kernel, n_batch, n_workers),
        out_type=jax.ShapeDtypeStruct((n_batch,), jnp.float32),
        mesh=mesh,
        scratch_types=[
            pltpu.VMEM((2 * bpw,), jnp.int32),
            pltpu.VMEM((bpw + HALF,), jnp.int32),
            pltpu.VMEM((bpw + HALF,), jnp.int32),
            pltpu.VMEM((bpw + HALF,), jnp.float32),
            pltpu.VMEM((SLOTS * HALF, DIM, WIN), jnp.float32),
            pltpu.VMEM((SLOTS * HALF, DIM, WIN), jnp.float32),
            pltpu.SemaphoreType.DMA,
            pltpu.SemaphoreType.DMA,
            pltpu.SemaphoreType.DMA,
            pltpu.SemaphoreType.DMA,
            pltpu.SemaphoreType.DMA,
            pltpu.SemaphoreType.DMA,
        ],
        compiler_params=pltpu.CompilerParams(needs_layout_passes=False),
    )
    return run(ij_flat, pt, mt)


# speedup vs baseline: 1.0045x; 1.0045x over previous
import functools

import jax
import jax.numpy as jnp
from jax import lax
from jax.experimental import pallas as pl
from jax.experimental.pallas import tpu as pltpu
from jax.experimental.pallas import tpu_sc as plsc

DIM = 16
ALPHA = 0.001
LANES = 16
WIN = 128
HALF = 8


def _dot_kernel(n_batch, n_rows, n_workers, ij_hbm, pt_hbm, mt_hbm, out_hbm,
                ij_v, iv_v, jv_v, out_v, pw, mw,
                sem_pa, sem_ma, sem_pb, sem_mb):
    bpw = n_batch // n_workers
    n_blocks = bpw // LANES
    wid = lax.axis_index("s") * 2 + lax.axis_index("c")
    base = wid * bpw

    pltpu.sync_copy(ij_hbm.at[pl.ds(2 * base, 2 * bpw)], ij_v)

    def unzip_block(blk, _):
        b0 = blk * LANES
        flat = 2 * (b0 + lax.iota(jnp.int32, LANES))
        iv_v[pl.ds(b0, LANES)] = plsc.load_gather(ij_v, [flat])
        jv_v[pl.ds(b0, LANES)] = plsc.load_gather(ij_v, [flat + 1])
        return 0

    lax.fori_loop(0, n_blocks, unzip_block, 0, unroll=4)

    def win_off(idx_scalar):
        return pl.multiple_of((idx_scalar >> 7) << 7, WIN)

    rows = lax.iota(jnp.int32, LANES)

    def fire_half(b0, off, slot, sp, sm):
        iv = iv_v[pl.ds(b0, LANES)]
        jv = jv_v[pl.ds(b0, LANES)]
        copies = []
        for u in range(HALF):
            l = off + u
            ci = win_off(iv[l])
            cj = win_off(jv[l])
            copies.append(pltpu.async_copy(
                pt_hbm.at[:, pl.ds(ci, WIN)], pw.at[slot * HALF + u], sp))
            copies.append(pltpu.async_copy(
                mt_hbm.at[:, pl.ds(cj, WIN)], mw.at[slot * HALF + u], sm))
        return copies

    def drain_half(slot, sp, sm):
        for u in range(HALF):
            pltpu.make_async_copy(
                pt_hbm.at[:, pl.ds(0, WIN)], pw.at[slot * HALF + u], sp).wait()
            pltpu.make_async_copy(
                mt_hbm.at[:, pl.ds(0, WIN)], mw.at[slot * HALF + u], sm).wait()

    def compute_half(b0, off, slot, res):
        iv = iv_v[pl.ds(b0, LANES)]
        jv = jv_v[pl.ds(b0, LANES)]
        for u in range(HALF):
            l = off + u
            oi = iv[l] - win_off(iv[l])
            oj = jv[l] - win_off(jv[l])
            pvec = plsc.load_gather(
                pw.at[slot * HALF + u], [rows, jnp.broadcast_to(oi, (LANES,))])
            mvec = plsc.load_gather(
                mw.at[slot * HALF + u], [rows, jnp.broadcast_to(oj, (LANES,))])
            s = jnp.sum(pvec * mvec)
            res = jnp.where(rows == l, s, res)
        return res

    def block_body(blk, _):
        b0 = blk * LANES
        copies_b = fire_half(b0, HALF, 1, sem_pb, sem_mb)
        drain_half(0, sem_pa, sem_ma)
        res = compute_half(b0, 0, 0, jnp.zeros((LANES,), jnp.float32))

        @pl.when(blk + 1 < n_blocks)
        def _():
            fire_half(b0 + LANES, 0, 0, sem_pa, sem_ma)

        for cp in copies_b:
            cp.wait()
        res = compute_half(b0, HALF, 1, res)
        out_v[pl.ds(b0, LANES)] = res * jnp.float32(ALPHA)
        return 0

    fire_half(0, 0, 0, sem_pa, sem_ma)
    lax.fori_loop(0, n_blocks, block_body, 0)

    pltpu.sync_copy(out_v, out_hbm.at[pl.ds(base, bpw)])


def kernel(ij, P, M):
    ij_flat = ij.astype(jnp.int32).reshape(-1)
    pt = P.T
    mt = M.T
    n_batch = ij.shape[0]
    n_rows = P.shape[0]
    info = plsc.get_sparse_core_info()
    n_workers = info.num_cores * info.num_subcores
    bpw = n_batch // n_workers

    mesh = plsc.VectorSubcoreMesh(core_axis_name="c", subcore_axis_name="s")
    run = pl.kernel(
        functools.partial(_dot_kernel, n_batch, n_rows, n_workers),
        out_type=jax.ShapeDtypeStruct((n_batch,), jnp.float32),
        mesh=mesh,
        scratch_types=[
            pltpu.VMEM((2 * bpw,), jnp.int32),
            pltpu.VMEM((bpw,), jnp.int32),
            pltpu.VMEM((bpw,), jnp.int32),
            pltpu.VMEM((bpw,), jnp.float32),
            pltpu.VMEM((2 * HALF, DIM, WIN), jnp.float32),
            pltpu.VMEM((2 * HALF, DIM, WIN), jnp.float32),
            pltpu.SemaphoreType.DMA,
            pltpu.SemaphoreType.DMA,
            pltpu.SemaphoreType.DMA,
            pltpu.SemaphoreType.DMA,
        ],
        compiler_params=pltpu.CompilerParams(needs_layout_passes=False),
    )
    return run(ij_flat, pt, mt)
